# speedup vs baseline: 2.4477x; 2.4477x over previous
import jax
import jax.numpy as jnp
from jax import lax
from jax.experimental import pallas as pl
from jax.experimental.pallas import tpu as pltpu
from jax.experimental.pallas import tpu_sc as plsc

NB = 10000
B = 16384
NC = 2
NS = 16
NW = NC * NS
CHUNK = B // NW
L = 16
IDXW = 128
NG = CHUNK // IDXW


def _sc_body(fw_hbm, pq_hbm, tab_hbm, out_hbm,
             fw_v, pq_v, i0, i1, i2, i3, vals_v, out_v, sem):
    idx_refs = (i0, i1, i2, i3)
    wid = lax.axis_index("s") * NC + lax.axis_index("c")
    base = wid * CHUNK

    cp_fw = pltpu.async_copy(fw_hbm.at[pl.ds(base, CHUNK)], fw_v, sem)
    cp_pq = pltpu.async_copy(pq_hbm.at[pl.ds(base, CHUNK)], pq_v, sem)
    cp_fw.wait()
    cp_pq.wait()

    for j in range(CHUNK // L):
        fw = fw_v[pl.ds(j * L, L)]
        pq = pq_v[pl.ds(j * L, L)]
        idx_refs[j // (IDXW // L)][pl.ds((j % (IDXW // L)) * L, L)] = fw * NB + pq

    gathers = [
        pltpu.async_copy(tab_hbm.at[idx_refs[g]],
                         vals_v.at[pl.ds(g * IDXW, IDXW)], sem)
        for g in range(NG)
    ]
    for c in gathers:
        c.wait()

    for j in range(CHUNK // L):
        x = vals_v[pl.ds(j * L, L)]
        out_v[pl.ds(j * L, L)] = 1.0 / (1.0 + jnp.exp(-x))

    pltpu.sync_copy(out_v, out_hbm.at[pl.ds(base, CHUNK)])


def kernel(z, fromWho, pourQui, embeds):
    fw = fromWho.reshape(-1).astype(jnp.int32)
    pq = pourQui.reshape(-1).astype(jnp.int32)
    tab = embeds.reshape(-1)

    mesh = plsc.VectorSubcoreMesh(core_axis_name="c", subcore_axis_name="s")
    k = pl.kernel(
        _sc_body,
        out_type=jax.ShapeDtypeStruct((B,), jnp.float32),
        mesh=mesh,
        scratch_types=[
            pltpu.VMEM((CHUNK,), jnp.int32),
            pltpu.VMEM((CHUNK,), jnp.int32),
            pltpu.VMEM((IDXW,), jnp.int32),
            pltpu.VMEM((IDXW,), jnp.int32),
            pltpu.VMEM((IDXW,), jnp.int32),
            pltpu.VMEM((IDXW,), jnp.int32),
            pltpu.VMEM((CHUNK,), jnp.float32),
            pltpu.VMEM((CHUNK,), jnp.float32),
            pltpu.SemaphoreType.DMA,
        ],
    )
    return k(fw, pq, tab)
